# Initial kernel scaffold; baseline (speedup 1.0000x reference)
import jax
import jax.numpy as jnp
from jax.experimental import pallas as pl


def kernel(H, edge_index, time_idx, Wp, bp, Wmsg, Wq, Wk, Wv, Wo, dbias):
    raise NotImplementedError("write your pallas kernel here")


# speedup vs baseline: 2.4902x; 2.4902x over previous
import functools

import jax
import jax.numpy as jnp
import numpy as np
from jax import lax
from jax.experimental import pallas as pl
from jax.experimental.pallas import tpu as pltpu
from jax.experimental.pallas import tpu_sc as plsc

T, N, E = 8, 10000, 160000
D, L, HEADS, W = 128, 2, 4, 8
DH = D // HEADS
TAU, VTH = 0.95, 1.0
TN = T * N

NC, NS = 2, 16
CH = 80
HC = CH // 2
EPAD = NS * CH * 128
NROWS = 10240
RPT = NROWS // NS
OUTR = 624
TPC = T // NC

BN = 400
BM = 2000


def _proj_body(h_ref, wp_ref, bp_ref, o_ref):
    o_ref[...] = (
        jnp.dot(h_ref[...], wp_ref[...], preferred_element_type=jnp.float32)
        + bp_ref[...]
    )


_proj = pl.pallas_call(
    _proj_body,
    grid=(TN // BM,),
    in_specs=[
        pl.BlockSpec((BM, D), lambda i: (i, 0)),
        pl.BlockSpec((D, D), lambda i: (0, 0)),
        pl.BlockSpec((1, D), lambda i: (0, 0)),
    ],
    out_specs=pl.BlockSpec((BM, D), lambda i: (i, 0)),
    out_shape=jax.ShapeDtypeStruct((TN, D), jnp.float32),
    compiler_params=pltpu.CompilerParams(dimension_semantics=("parallel",)),
)


def _layer_body(x_ref, inp_ref, agg_ref, wm_ref, wq_ref, wk_ref, wv_ref,
                wo_ref, b_ref, xo_ref, s_ref, in_ref, z_ref):
    x = x_ref[...]
    inp = inp_ref[...]

    def mm(a3, w_ref):
        return jnp.dot(
            a3.reshape(T * BN, D), w_ref[...],
            preferred_element_type=jnp.float32,
        ).reshape(T, BN, D)

    m = mm(agg_ref[...], wm_ref)
    q = mm(inp, wq_ref)
    k = mm(inp, wk_ref)
    v = mm(inp, wv_ref)

    def b16(u):
        return u.astype(jnp.bfloat16).astype(jnp.float32)

    qb = b16(q)
    kb = b16(k)
    vb = b16(v)

    hh = lax.broadcasted_iota(jnp.int32, (HEADS, D), 0)
    jj = lax.broadcasted_iota(jnp.int32, (HEADS, D), 1) // DH
    emat = (hh == jj).astype(jnp.float32)
    rsq = np.sqrt(DH).astype(np.float32)

    qbT = [jnp.transpose(qb[t]) for t in range(T)]
    kbT = [jnp.transpose(kb[t]) for t in range(T)]

    def head_sum(p):
        for _ in range(3):
            r = p.reshape(p.shape[0] // 2, 2, BN)
            p = r[:, 0] + r[:, 1]
        r4 = p.reshape(HEADS, 4, BN)
        return ((r4[:, 0] + r4[:, 1]) + r4[:, 2]) + r4[:, 3]

    avs = []
    for t in range(T):
        sb = []
        for s in range(t + 1):
            sh = head_sum(qbT[t] * kbT[s])
            sc = lax.dot_general(
                sh, emat, (((0,), (0,)), ((), ())),
                precision=lax.Precision.HIGHEST,
                preferred_element_type=jnp.float32)
            sb.append(sc / rsq + b_ref[t, s][None, :])
        mx = sb[t]
        for s in range(t):
            mx = jnp.maximum(mx, sb[s])
        es = [jnp.exp(e - mx) for e in sb]

        slots = [es[s] if s <= t else None for s in range(T)]
        n = T
        while n > 1:
            half = n // 2
            nxt = []
            for i in range(half):
                a, b = slots[i], slots[i + half]
                nxt.append(a if b is None else (b if a is None else a + b))
            slots = nxt
            n = half
        den = slots[0]

        terms = [b16(es[s] / den) * vb[s] if s <= t else None
                 for s in range(T)]
        while len(terms) > 1:
            nxt = []
            for i in range(0, len(terms), 2):
                a, b = terms[i], terms[i + 1]
                nxt.append(a if b is None else (b if a is None else a + b))
            terms = nxt
        avs.append(terms[0])
    av = jnp.stack(avs, axis=0)
    y = mm(av, wo_ref) + m

    vm = jnp.zeros((BN, D), jnp.float32)
    zacc = jnp.zeros((BN, D), jnp.float32)
    for t in range(T):
        yt = y[t]
        vm = TAU * vm + yt
        st = (vm >= VTH).astype(jnp.float32)
        vm = vm - st * VTH
        xo = x[t] + yt
        xo_ref[t] = xo
        s_ref[t] = st
        in_ref[t] = xo + st
        zacc = zacc + xo
    z_ref[...] = zacc * (1.0 / T)


_layer = pl.pallas_call(
    _layer_body,
    grid=(N // BN,),
    in_specs=[
        pl.BlockSpec((T, BN, D), lambda i: (0, i, 0)),
        pl.BlockSpec((T, BN, D), lambda i: (0, i, 0)),
        pl.BlockSpec((T, BN, D), lambda i: (0, i, 0)),
        pl.BlockSpec((D, D), lambda i: (0, 0)),
        pl.BlockSpec((D, D), lambda i: (0, 0)),
        pl.BlockSpec((D, D), lambda i: (0, 0)),
        pl.BlockSpec((D, D), lambda i: (0, 0)),
        pl.BlockSpec((D, D), lambda i: (0, 0)),
        pl.BlockSpec((T, T, D), lambda i: (0, 0, 0)),
    ],
    out_specs=[
        pl.BlockSpec((T, BN, D), lambda i: (0, i, 0)),
        pl.BlockSpec((T, BN, D), lambda i: (0, i, 0)),
        pl.BlockSpec((T, BN, D), lambda i: (0, i, 0)),
        pl.BlockSpec((BN, D), lambda i: (i, 0)),
    ],
    out_shape=[
        jax.ShapeDtypeStruct((T, N, D), jnp.float32),
        jax.ShapeDtypeStruct((T, N, D), jnp.float32),
        jax.ShapeDtypeStruct((T, N, D), jnp.float32),
        jax.ShapeDtypeStruct((N, D), jnp.float32),
    ],
    compiler_params=pltpu.CompilerParams(dimension_semantics=("parallel",)),
)


def _seg_body(inp_hbm, srct_hbm, dstb_hbm, out_hbm,
              srcv, dstv, buf0, buf1, zbuf, aggsh, sem0, sem1):
    c = lax.axis_index("c")
    s = lax.axis_index("s")

    zv = jnp.zeros((16,), jnp.float32)
    for i in range(16):
        for j in range(D // 16):
            zbuf[i, pl.ds(j * 16, 16)] = zv

    def gather_start(g, buf, sem):
        pltpu.make_async_copy(inp_hbm.at[srcv.at[g]], buf, sem).start()

    def gather_wait(buf, sem):
        pltpu.make_async_copy(inp_hbm.at[srcv.at[0]], buf, sem).wait()

    def per_t(ti, carry):
        t = c * TPC + ti

        def zloop(i, _):
            pltpu.sync_copy(zbuf, aggsh.at[pl.ds(s * RPT + i * 16, 16)])
            return 0
        lax.fori_loop(0, RPT // 16, zloop, 0)
        plsc.subcore_barrier()

        for h in range(2):
            pltpu.sync_copy(srct_hbm.at[t, s, pl.ds(h * HC, HC)], srcv)
            pltpu.sync_copy(dstb_hbm.at[s, pl.ds(h * HC, HC)], dstv)
            gather_start(0, buf0, sem0)

            def chunk(i, _):
                g0 = 2 * i
                gather_wait(buf0, sem0)
                gather_start(g0 + 1, buf1, sem1)
                pltpu.sync_copy(buf0, aggsh.at[dstv.at[g0]], add=True)
                gather_wait(buf1, sem1)
                gather_start(jnp.minimum(g0 + 2, HC - 1), buf0, sem0)
                pltpu.sync_copy(buf1, aggsh.at[dstv.at[g0 + 1]], add=True)
                return 0
            lax.fori_loop(0, HC // 2, chunk, 0)
            gather_wait(buf0, sem0)
        plsc.subcore_barrier()

        pltpu.sync_copy(
            aggsh.at[pl.ds(s * OUTR, OUTR)],
            out_hbm.at[pl.ds(t * N + s * OUTR, OUTR)],
        )

        @pl.when(s == NS - 1)
        def _tail():
            pltpu.sync_copy(
                aggsh.at[pl.ds(NS * OUTR, N - NS * OUTR)],
                out_hbm.at[pl.ds(t * N + NS * OUTR, N - NS * OUTR)],
            )
        plsc.subcore_barrier()
        return carry

    lax.fori_loop(0, TPC, per_t, 0)


@functools.cache
def _get_segsum():
    return pl.kernel(
        _seg_body,
        out_type=jax.ShapeDtypeStruct((TN, D), jnp.float32),
        mesh=plsc.VectorSubcoreMesh(core_axis_name="c", subcore_axis_name="s"),
        scratch_types=[
            pltpu.VMEM((HC, 128), jnp.int32),
            pltpu.VMEM((HC, 128), jnp.int32),
            pltpu.VMEM((128, D), jnp.float32),
            pltpu.VMEM((128, D), jnp.float32),
            pltpu.VMEM((16, D), jnp.float32),
            pltpu.VMEM_SHARED((NROWS, D), jnp.float32),
            pltpu.SemaphoreType.DMA,
            pltpu.SemaphoreType.DMA,
        ],
    )


def _head_bias(dbias_l):
    delta = np.arange(T)[:, None] - np.arange(T)[None, :]
    dl = jnp.asarray(np.clip(delta, 0, W - 1), dtype=jnp.int32)
    bias_tsh = dbias_l[:, dl]
    lane_head = jnp.asarray(np.arange(D) // DH, dtype=jnp.int32)
    return jnp.transpose(bias_tsh[lane_head], (1, 2, 0))


def kernel(H, edge_index, time_idx, Wp, bp, Wmsg, Wq, Wk, Wv, Wo, dbias):
    del time_idx
    Hf = H.reshape(TN, D)
    x = _proj(Hf, Wp, bp.reshape(1, D))

    pad = EPAD - E
    srcp = jnp.concatenate(
        [edge_index[0], jnp.zeros((pad,), jnp.int32)]).reshape(NS, CH, 128)
    dstp = jnp.concatenate(
        [edge_index[1], jnp.full((pad,), N, jnp.int32)]).reshape(NS, CH, 128)
    srct = (srcp[None] +
            (jnp.arange(T, dtype=jnp.int32) * N)[:, None, None, None])

    inp = x
    spikes = []
    xo = z = None
    for l in range(L):
        agg = _get_segsum()(inp, srct, dstp)
        xo, s_l, inext, z = _layer(
            x.reshape(T, N, D), inp.reshape(T, N, D), agg.reshape(T, N, D),
            Wmsg[l], Wq[l], Wk[l], Wv[l], Wo[l], _head_bias(dbias[l]))
        spikes.append(s_l)
        x = xo.reshape(TN, D)
        inp = inext.reshape(TN, D)
    return z, xo, jnp.stack(spikes, axis=0)


# speedup vs baseline: 2.5617x; 1.0287x over previous
import functools

import jax
import jax.numpy as jnp
import numpy as np
from jax import lax
from jax.experimental import pallas as pl
from jax.experimental.pallas import tpu as pltpu
from jax.experimental.pallas import tpu_sc as plsc

T, N, E = 8, 10000, 160000
D, L, HEADS, W = 128, 2, 4, 8
DH = D // HEADS
TAU, VTH = 0.95, 1.0
TN = T * N

NC, NS = 2, 16
CH = 80
HC = CH // 2
EPAD = NS * CH * 128
NROWS = 10240
RPT = NROWS // NS
OUTR = 624
TPC = T // NC

BN = 400
BM = 2000


def _proj_body(h_ref, wp_ref, bp_ref, o_ref):
    o_ref[...] = (
        jnp.dot(h_ref[...], wp_ref[...], preferred_element_type=jnp.float32)
        + bp_ref[...]
    )


_proj = pl.pallas_call(
    _proj_body,
    grid=(TN // BM,),
    in_specs=[
        pl.BlockSpec((BM, D), lambda i: (i, 0)),
        pl.BlockSpec((D, D), lambda i: (0, 0)),
        pl.BlockSpec((1, D), lambda i: (0, 0)),
    ],
    out_specs=pl.BlockSpec((BM, D), lambda i: (i, 0)),
    out_shape=jax.ShapeDtypeStruct((TN, D), jnp.float32),
    compiler_params=pltpu.CompilerParams(dimension_semantics=("parallel",)),
)


def _layer_body(x_ref, inp_ref, agg_ref, wm_ref, wq_ref, wk_ref, wv_ref,
                wo_ref, b_ref, xo_ref, s_ref, in_ref, z_ref):
    x = x_ref[...]
    inp = inp_ref[...]

    def mm(a3, w_ref):
        return jnp.dot(
            a3.reshape(T * BN, D), w_ref[...],
            preferred_element_type=jnp.float32,
        ).reshape(T, BN, D)

    m = mm(agg_ref[...], wm_ref)
    q = mm(inp, wq_ref)
    k = mm(inp, wk_ref)
    v = mm(inp, wv_ref)

    def b16(u):
        return u.astype(jnp.bfloat16).astype(jnp.float32)

    qb = b16(q)
    kb = b16(k)
    vb = b16(v)

    ri = lax.broadcasted_iota(jnp.int32, (D, D), 0) // DH
    ci = lax.broadcasted_iota(jnp.int32, (D, D), 1) // DH
    hsum = (ri == ci).astype(jnp.float32)
    rsq = np.sqrt(DH).astype(np.float32)

    avs = []
    for t in range(T):
        sb = []
        for s in range(t + 1):
            prod = qb[t] * kb[s]
            sc = jnp.dot(prod, hsum, preferred_element_type=jnp.float32,
                         precision=lax.Precision.HIGHEST)
            sb.append(sc / rsq + b_ref[t, s][None, :])
        mx = sb[t]
        for s in range(t):
            mx = jnp.maximum(mx, sb[s])
        es = [jnp.exp(e - mx) for e in sb]

        slots = [es[s] if s <= t else None for s in range(T)]
        n = T
        while n > 1:
            half = n // 2
            nxt = []
            for i in range(half):
                a, b = slots[i], slots[i + half]
                nxt.append(a if b is None else (b if a is None else a + b))
            slots = nxt
            n = half
        den = slots[0]

        terms = [b16(es[s] / den) * vb[s] if s <= t else None
                 for s in range(T)]
        while len(terms) > 1:
            nxt = []
            for i in range(0, len(terms), 2):
                a, b = terms[i], terms[i + 1]
                nxt.append(a if b is None else (b if a is None else a + b))
            terms = nxt
        avs.append(terms[0])
    av = jnp.stack(avs, axis=0)
    y = mm(av, wo_ref) + m

    vm = jnp.zeros((BN, D), jnp.float32)
    zacc = jnp.zeros((BN, D), jnp.float32)
    for t in range(T):
        yt = y[t]
        vm = TAU * vm + yt
        st = (vm >= VTH).astype(jnp.float32)
        vm = vm - st * VTH
        xo = x[t] + yt
        xo_ref[t] = xo
        s_ref[t] = st
        in_ref[t] = xo + st
        zacc = zacc + xo
    z_ref[...] = zacc * (1.0 / T)


_layer = pl.pallas_call(
    _layer_body,
    grid=(N // BN,),
    in_specs=[
        pl.BlockSpec((T, BN, D), lambda i: (0, i, 0)),
        pl.BlockSpec((T, BN, D), lambda i: (0, i, 0)),
        pl.BlockSpec((T, BN, D), lambda i: (0, i, 0)),
        pl.BlockSpec((D, D), lambda i: (0, 0)),
        pl.BlockSpec((D, D), lambda i: (0, 0)),
        pl.BlockSpec((D, D), lambda i: (0, 0)),
        pl.BlockSpec((D, D), lambda i: (0, 0)),
        pl.BlockSpec((D, D), lambda i: (0, 0)),
        pl.BlockSpec((T, T, D), lambda i: (0, 0, 0)),
    ],
    out_specs=[
        pl.BlockSpec((T, BN, D), lambda i: (0, i, 0)),
        pl.BlockSpec((T, BN, D), lambda i: (0, i, 0)),
        pl.BlockSpec((T, BN, D), lambda i: (0, i, 0)),
        pl.BlockSpec((BN, D), lambda i: (i, 0)),
    ],
    out_shape=[
        jax.ShapeDtypeStruct((T, N, D), jnp.float32),
        jax.ShapeDtypeStruct((T, N, D), jnp.float32),
        jax.ShapeDtypeStruct((T, N, D), jnp.float32),
        jax.ShapeDtypeStruct((N, D), jnp.float32),
    ],
    compiler_params=pltpu.CompilerParams(dimension_semantics=("parallel",)),
)


def _seg_body(inp_hbm, srct_hbm, dstb_hbm, out_hbm,
              srcv, dstv, buf0, buf1, zbuf, aggsh, sem0, sem1):
    c = lax.axis_index("c")
    s = lax.axis_index("s")

    zv = jnp.zeros((16,), jnp.float32)
    for i in range(16):
        for j in range(D // 16):
            zbuf[i, pl.ds(j * 16, 16)] = zv

    def gather_start(g, buf, sem):
        pltpu.make_async_copy(inp_hbm.at[srcv.at[g]], buf, sem).start()

    def gather_wait(buf, sem):
        pltpu.make_async_copy(inp_hbm.at[srcv.at[0]], buf, sem).wait()

    def per_t(ti, carry):
        t = c * TPC + ti

        def zloop(i, _):
            pltpu.sync_copy(zbuf, aggsh.at[pl.ds(s * RPT + i * 16, 16)])
            return 0
        lax.fori_loop(0, RPT // 16, zloop, 0)
        plsc.subcore_barrier()

        for h in range(2):
            pltpu.sync_copy(srct_hbm.at[t, s, pl.ds(h * HC, HC)], srcv)
            pltpu.sync_copy(dstb_hbm.at[s, pl.ds(h * HC, HC)], dstv)
            gather_start(0, buf0, sem0)

            def chunk(i, _):
                g0 = 2 * i
                gather_wait(buf0, sem0)
                gather_start(g0 + 1, buf1, sem1)
                pltpu.sync_copy(buf0, aggsh.at[dstv.at[g0]], add=True)
                gather_wait(buf1, sem1)
                gather_start(jnp.minimum(g0 + 2, HC - 1), buf0, sem0)
                pltpu.sync_copy(buf1, aggsh.at[dstv.at[g0 + 1]], add=True)
                return 0
            lax.fori_loop(0, HC // 2, chunk, 0)
            gather_wait(buf0, sem0)
        plsc.subcore_barrier()

        pltpu.sync_copy(
            aggsh.at[pl.ds(s * OUTR, OUTR)],
            out_hbm.at[pl.ds(t * N + s * OUTR, OUTR)],
        )

        @pl.when(s == NS - 1)
        def _tail():
            pltpu.sync_copy(
                aggsh.at[pl.ds(NS * OUTR, N - NS * OUTR)],
                out_hbm.at[pl.ds(t * N + NS * OUTR, N - NS * OUTR)],
            )
        plsc.subcore_barrier()
        return carry

    lax.fori_loop(0, TPC, per_t, 0)


@functools.cache
def _get_segsum():
    return pl.kernel(
        _seg_body,
        out_type=jax.ShapeDtypeStruct((TN, D), jnp.float32),
        mesh=plsc.VectorSubcoreMesh(core_axis_name="c", subcore_axis_name="s"),
        scratch_types=[
            pltpu.VMEM((HC, 128), jnp.int32),
            pltpu.VMEM((HC, 128), jnp.int32),
            pltpu.VMEM((128, D), jnp.float32),
            pltpu.VMEM((128, D), jnp.float32),
            pltpu.VMEM((16, D), jnp.float32),
            pltpu.VMEM_SHARED((NROWS, D), jnp.float32),
            pltpu.SemaphoreType.DMA,
            pltpu.SemaphoreType.DMA,
        ],
    )


def _head_bias(dbias_l):
    delta = np.arange(T)[:, None] - np.arange(T)[None, :]
    dl = jnp.asarray(np.clip(delta, 0, W - 1), dtype=jnp.int32)
    bias_tsh = dbias_l[:, dl]
    lane_head = jnp.asarray(np.arange(D) // DH, dtype=jnp.int32)
    return jnp.transpose(bias_tsh[lane_head], (1, 2, 0))


def kernel(H, edge_index, time_idx, Wp, bp, Wmsg, Wq, Wk, Wv, Wo, dbias):
    del time_idx
    Hf = H.reshape(TN, D)
    x = _proj(Hf, Wp, bp.reshape(1, D))

    pad = EPAD - E
    srcp = jnp.concatenate(
        [edge_index[0], jnp.zeros((pad,), jnp.int32)]).reshape(NS, CH, 128)
    dstp = jnp.concatenate(
        [edge_index[1], jnp.full((pad,), N, jnp.int32)]).reshape(NS, CH, 128)
    srct = (srcp[None] +
            (jnp.arange(T, dtype=jnp.int32) * N)[:, None, None, None])

    inp = x
    spikes = []
    xo = z = None
    for l in range(L):
        agg = _get_segsum()(inp, srct, dstp)
        xo, s_l, inext, z = _layer(
            x.reshape(T, N, D), inp.reshape(T, N, D), agg.reshape(T, N, D),
            Wmsg[l], Wq[l], Wk[l], Wv[l], Wo[l], _head_bias(dbias[l]))
        spikes.append(s_l)
        x = xo.reshape(TN, D)
        inp = inext.reshape(TN, D)
    return z, xo, jnp.stack(spikes, axis=0)


# speedup vs baseline: 2.5716x; 1.0038x over previous
import functools

import jax
import jax.numpy as jnp
import numpy as np
from jax import lax
from jax.experimental import pallas as pl
from jax.experimental.pallas import tpu as pltpu
from jax.experimental.pallas import tpu_sc as plsc

T, N, E = 8, 10000, 160000
D, L, HEADS, W = 128, 2, 4, 8
DH = D // HEADS
TAU, VTH = 0.95, 1.0
TN = T * N

NC, NS = 2, 16
CH = 80
HC = CH // 2
EPAD = NS * CH * 128
NROWS = 10240
RPT = NROWS // NS
ZR = 32
OUTR = 624
TPC = T // NC

BN = 400
BM = 2000


def _proj_body(h_ref, wp_ref, bp_ref, o_ref):
    o_ref[...] = (
        jnp.dot(h_ref[...], wp_ref[...], preferred_element_type=jnp.float32)
        + bp_ref[...]
    )


_proj = pl.pallas_call(
    _proj_body,
    grid=(TN // BM,),
    in_specs=[
        pl.BlockSpec((BM, D), lambda i: (i, 0)),
        pl.BlockSpec((D, D), lambda i: (0, 0)),
        pl.BlockSpec((1, D), lambda i: (0, 0)),
    ],
    out_specs=pl.BlockSpec((BM, D), lambda i: (i, 0)),
    out_shape=jax.ShapeDtypeStruct((TN, D), jnp.float32),
    compiler_params=pltpu.CompilerParams(dimension_semantics=("parallel",)),
)


def _layer_body(x_ref, inp_ref, agg_ref, wm_ref, wq_ref, wk_ref, wv_ref,
                wo_ref, b_ref, xo_ref, s_ref, in_ref, z_ref):
    x = x_ref[...]
    inp = inp_ref[...]

    def mm(a3, w_ref):
        return jnp.dot(
            a3.reshape(T * BN, D), w_ref[...],
            preferred_element_type=jnp.float32,
        ).reshape(T, BN, D)

    m = mm(agg_ref[...], wm_ref)
    q = mm(inp, wq_ref)
    k = mm(inp, wk_ref)
    v = mm(inp, wv_ref)

    def b16(u):
        return u.astype(jnp.bfloat16).astype(jnp.float32)

    qb = b16(q)
    kb = b16(k)
    vb = b16(v)

    ri = lax.broadcasted_iota(jnp.int32, (D, D), 0) // DH
    ci = lax.broadcasted_iota(jnp.int32, (D, D), 1) // DH
    hsum = (ri == ci).astype(jnp.float32)
    rsq = np.sqrt(DH).astype(np.float32)

    avs = []
    for t in range(T):
        sb = []
        for s in range(t + 1):
            prod = qb[t] * kb[s]
            sc = jnp.dot(prod, hsum, preferred_element_type=jnp.float32,
                         precision=lax.Precision.HIGHEST)
            sb.append(sc / rsq + b_ref[t, s][None, :])
        mx = sb[t]
        for s in range(t):
            mx = jnp.maximum(mx, sb[s])
        es = [jnp.exp(e - mx) for e in sb]

        slots = [es[s] if s <= t else None for s in range(T)]
        n = T
        while n > 1:
            half = n // 2
            nxt = []
            for i in range(half):
                a, b = slots[i], slots[i + half]
                nxt.append(a if b is None else (b if a is None else a + b))
            slots = nxt
            n = half
        den = slots[0]

        terms = [b16(es[s] / den) * vb[s] if s <= t else None
                 for s in range(T)]
        while len(terms) > 1:
            nxt = []
            for i in range(0, len(terms), 2):
                a, b = terms[i], terms[i + 1]
                nxt.append(a if b is None else (b if a is None else a + b))
            terms = nxt
        avs.append(terms[0])
    av = jnp.stack(avs, axis=0)
    y = mm(av, wo_ref) + m

    vm = jnp.zeros((BN, D), jnp.float32)
    zacc = jnp.zeros((BN, D), jnp.float32)
    for t in range(T):
        yt = y[t]
        vm = TAU * vm + yt
        st = (vm >= VTH).astype(jnp.float32)
        vm = vm - st * VTH
        xo = x[t] + yt
        xo_ref[t] = xo
        s_ref[t] = st
        in_ref[t] = xo + st
        zacc = zacc + xo
    z_ref[...] = zacc * (1.0 / T)


_layer = pl.pallas_call(
    _layer_body,
    grid=(N // BN,),
    in_specs=[
        pl.BlockSpec((T, BN, D), lambda i: (0, i, 0)),
        pl.BlockSpec((T, BN, D), lambda i: (0, i, 0)),
        pl.BlockSpec((T, BN, D), lambda i: (0, i, 0)),
        pl.BlockSpec((D, D), lambda i: (0, 0)),
        pl.BlockSpec((D, D), lambda i: (0, 0)),
        pl.BlockSpec((D, D), lambda i: (0, 0)),
        pl.BlockSpec((D, D), lambda i: (0, 0)),
        pl.BlockSpec((D, D), lambda i: (0, 0)),
        pl.BlockSpec((T, T, D), lambda i: (0, 0, 0)),
    ],
    out_specs=[
        pl.BlockSpec((T, BN, D), lambda i: (0, i, 0)),
        pl.BlockSpec((T, BN, D), lambda i: (0, i, 0)),
        pl.BlockSpec((T, BN, D), lambda i: (0, i, 0)),
        pl.BlockSpec((BN, D), lambda i: (i, 0)),
    ],
    out_shape=[
        jax.ShapeDtypeStruct((T, N, D), jnp.float32),
        jax.ShapeDtypeStruct((T, N, D), jnp.float32),
        jax.ShapeDtypeStruct((T, N, D), jnp.float32),
        jax.ShapeDtypeStruct((N, D), jnp.float32),
    ],
    compiler_params=pltpu.CompilerParams(dimension_semantics=("parallel",)),
)


def _seg_body(inp_hbm, srct_hbm, dstb_hbm, out_hbm,
              srcv, dstv, buf0, buf1, zbuf, aggsh,
              gs0, gs1, ss0, ss1, zs):
    c = lax.axis_index("c")
    s = lax.axis_index("s")

    zv = jnp.zeros((16,), jnp.float32)
    for i in range(ZR):
        for j in range(D // 16):
            zbuf[i, pl.ds(j * 16, 16)] = zv

    def gather_start(g, buf, sem):
        pltpu.make_async_copy(inp_hbm.at[srcv.at[g]], buf, sem).start()

    def gather_wait(buf, sem):
        pltpu.make_async_copy(inp_hbm.at[srcv.at[0]], buf, sem).wait()

    def scat_start(buf, g, sem):
        pltpu.async_copy(buf, aggsh.at[dstv.at[g]], sem, add=True)

    def scat_wait(buf, sem):
        pltpu.make_async_copy(buf, aggsh.at[dstv.at[0]], sem).wait()

    def per_t(ti, carry):
        t = c * TPC + ti

        def zstart(i, _):
            pltpu.make_async_copy(
                zbuf, aggsh.at[pl.ds(s * RPT + i * ZR, ZR)], zs).start()
            return 0
        lax.fori_loop(0, RPT // ZR, zstart, 0)

        def zdrain(i, _):
            pltpu.make_async_copy(
                zbuf, aggsh.at[pl.ds(s * RPT, ZR)], zs).wait()
            return 0
        lax.fori_loop(0, RPT // ZR, zdrain, 0)
        plsc.subcore_barrier()

        for h in range(2):
            pltpu.sync_copy(srct_hbm.at[t, s, pl.ds(h * HC, HC)], srcv)
            pltpu.sync_copy(dstb_hbm.at[s, pl.ds(h * HC, HC)], dstv)
            gather_start(0, buf0, gs0)

            def chunk(i, _):
                g0 = 2 * i
                gather_wait(buf0, gs0)
                scat_start(buf0, g0, ss0)

                @pl.when(i > 0)
                def _():
                    scat_wait(buf1, ss1)
                gather_start(g0 + 1, buf1, gs1)
                gather_wait(buf1, gs1)
                scat_start(buf1, g0 + 1, ss1)
                scat_wait(buf0, ss0)
                gather_start(jnp.minimum(g0 + 2, HC - 1), buf0, gs0)
                return 0
            lax.fori_loop(0, HC // 2, chunk, 0)
            scat_wait(buf1, ss1)
            gather_wait(buf0, gs0)
        plsc.subcore_barrier()

        pltpu.sync_copy(
            aggsh.at[pl.ds(s * OUTR, OUTR)],
            out_hbm.at[pl.ds(t * N + s * OUTR, OUTR)],
        )

        @pl.when(s == NS - 1)
        def _tail():
            pltpu.sync_copy(
                aggsh.at[pl.ds(NS * OUTR, N - NS * OUTR)],
                out_hbm.at[pl.ds(t * N + NS * OUTR, N - NS * OUTR)],
            )
        plsc.subcore_barrier()
        return carry

    lax.fori_loop(0, TPC, per_t, 0)


@functools.cache
def _get_segsum():
    return pl.kernel(
        _seg_body,
        out_type=jax.ShapeDtypeStruct((TN, D), jnp.float32),
        mesh=plsc.VectorSubcoreMesh(core_axis_name="c", subcore_axis_name="s"),
        scratch_types=[
            pltpu.VMEM((HC, 128), jnp.int32),
            pltpu.VMEM((HC, 128), jnp.int32),
            pltpu.VMEM((128, D), jnp.float32),
            pltpu.VMEM((128, D), jnp.float32),
            pltpu.VMEM((ZR, D), jnp.float32),
            pltpu.VMEM_SHARED((NROWS, D), jnp.float32),
            pltpu.SemaphoreType.DMA,
            pltpu.SemaphoreType.DMA,
            pltpu.SemaphoreType.DMA,
            pltpu.SemaphoreType.DMA,
            pltpu.SemaphoreType.DMA,
        ],
    )


def _head_bias(dbias_l):
    delta = np.arange(T)[:, None] - np.arange(T)[None, :]
    dl = jnp.asarray(np.clip(delta, 0, W - 1), dtype=jnp.int32)
    bias_tsh = dbias_l[:, dl]
    lane_head = jnp.asarray(np.arange(D) // DH, dtype=jnp.int32)
    return jnp.transpose(bias_tsh[lane_head], (1, 2, 0))


def kernel(H, edge_index, time_idx, Wp, bp, Wmsg, Wq, Wk, Wv, Wo, dbias):
    del time_idx
    Hf = H.reshape(TN, D)
    x = _proj(Hf, Wp, bp.reshape(1, D))

    pad = EPAD - E
    srcp = jnp.concatenate(
        [edge_index[0], jnp.zeros((pad,), jnp.int32)]).reshape(NS, CH, 128)
    dstp = jnp.concatenate(
        [edge_index[1], jnp.full((pad,), N, jnp.int32)]).reshape(NS, CH, 128)
    srct = (srcp[None] +
            (jnp.arange(T, dtype=jnp.int32) * N)[:, None, None, None])

    inp = x
    spikes = []
    xo = z = None
    for l in range(L):
        agg = _get_segsum()(inp, srct, dstp)
        xo, s_l, inext, z = _layer(
            x.reshape(T, N, D), inp.reshape(T, N, D), agg.reshape(T, N, D),
            Wmsg[l], Wq[l], Wk[l], Wv[l], Wo[l], _head_bias(dbias[l]))
        spikes.append(s_l)
        x = xo.reshape(TN, D)
        inp = inext.reshape(TN, D)
    return z, xo, jnp.stack(spikes, axis=0)
